# baseline (speedup 1.0000x reference)
import jax
import jax.numpy as jnp
from jax.experimental import pallas as pl

_NUM_CELLS = 8192
_CELL_DIM = 32
_BS = 1024
_LANE = 128


def _pc_argmax_kernel(x_ref, pc_ref, out_ref):
    s = jax.lax.dot_general(
        x_ref[...], pc_ref[...],
        dimension_numbers=(((1,), (1,)), ((), ())),
        preferred_element_type=jnp.float32,
    )
    nt = _NUM_CELLS // _LANE
    m = s[:, 0:_LANE]
    ti = jnp.zeros((_BS, _LANE), jnp.float32)
    for j in range(1, nt):
        sj = s[:, j * _LANE:(j + 1) * _LANE]
        g = sj > m
        m = jnp.maximum(m, sj)
        ti = jnp.where(g, jnp.float32(j), ti)
    lane = jax.lax.broadcasted_iota(jnp.int32, (_BS, _LANE), 1).astype(jnp.float32)
    full = ti * jnp.float32(_LANE) + lane
    rm = jnp.max(m, axis=1, keepdims=True)
    idx = jnp.min(jnp.where(m == rm, full, jnp.float32(_NUM_CELLS)), axis=1)
    out_ref[...] = idx.astype(jnp.int32)


def kernel(x, placeCells):
    states = jnp.reshape(x, (-1, _CELL_DIM))
    n = states.shape[0]
    return pl.pallas_call(
        _pc_argmax_kernel,
        grid=(n // _BS,),
        in_specs=[
            pl.BlockSpec((_BS, _CELL_DIM), lambda i: (i, 0)),
            pl.BlockSpec((_NUM_CELLS, _CELL_DIM), lambda i: (0, 0)),
        ],
        out_specs=pl.BlockSpec((_BS,), lambda i: (i,)),
        out_shape=jax.ShapeDtypeStruct((n,), jnp.int32),
    )(states, placeCells)


# speedup vs baseline: 1.0160x; 1.0160x over previous
import jax
import jax.numpy as jnp
from jax.experimental import pallas as pl

_NUM_CELLS = 8192
_CELL_DIM = 32
_BS = 512
_LANE = 128


def _pc_argmax_kernel(x_ref, pc_ref, out_ref):
    s = jax.lax.dot_general(
        x_ref[...], pc_ref[...],
        dimension_numbers=(((1,), (1,)), ((), ())),
        preferred_element_type=jnp.float32,
    )
    nt = _NUM_CELLS // _LANE
    m = s[:, 0:_LANE]
    ti = jnp.zeros((_BS, _LANE), jnp.float32)
    for j in range(1, nt):
        sj = s[:, j * _LANE:(j + 1) * _LANE]
        g = sj > m
        m = jnp.maximum(m, sj)
        ti = jnp.where(g, jnp.float32(j), ti)
    lane = jax.lax.broadcasted_iota(jnp.int32, (_BS, _LANE), 1).astype(jnp.float32)
    full = ti * jnp.float32(_LANE) + lane
    rm = jnp.max(m, axis=1, keepdims=True)
    idx = jnp.min(jnp.where(m == rm, full, jnp.float32(_NUM_CELLS)), axis=1)
    out_ref[...] = idx.astype(jnp.int32)


def kernel(x, placeCells):
    states = jnp.reshape(x, (-1, _CELL_DIM))
    n = states.shape[0]
    return pl.pallas_call(
        _pc_argmax_kernel,
        grid=(n // _BS,),
        in_specs=[
            pl.BlockSpec((_BS, _CELL_DIM), lambda i: (i, 0)),
            pl.BlockSpec((_NUM_CELLS, _CELL_DIM), lambda i: (0, 0)),
        ],
        out_specs=pl.BlockSpec((_BS,), lambda i: (i,)),
        out_shape=jax.ShapeDtypeStruct((n,), jnp.int32),
    )(states, placeCells)


# speedup vs baseline: 1.0624x; 1.0456x over previous
import jax
import jax.numpy as jnp
from jax.experimental import pallas as pl

_NUM_CELLS = 8192
_CELL_DIM = 32
_BS = 512
_LANE = 128


def _pc_argmax_kernel(x_ref, pc_ref, out_ref):
    xb = x_ref[...]
    nt = _NUM_CELLS // _LANE
    m = None
    ti = jnp.zeros((_BS, _LANE), jnp.float32)
    for j in range(nt):
        pcj = pc_ref[j * _LANE:(j + 1) * _LANE, :]
        sj = jax.lax.dot_general(
            xb, pcj,
            dimension_numbers=(((1,), (1,)), ((), ())),
            preferred_element_type=jnp.float32,
        )
        if j == 0:
            m = sj
        else:
            g = sj > m
            m = jnp.maximum(m, sj)
            ti = jnp.where(g, jnp.float32(j), ti)
    lane = jax.lax.broadcasted_iota(jnp.int32, (_BS, _LANE), 1).astype(jnp.float32)
    full = ti * jnp.float32(_LANE) + lane
    rm = jnp.max(m, axis=1, keepdims=True)
    idx = jnp.min(jnp.where(m == rm, full, jnp.float32(_NUM_CELLS)), axis=1)
    out_ref[...] = idx.astype(jnp.int32)


def kernel(x, placeCells):
    states = jnp.reshape(x, (-1, _CELL_DIM))
    n = states.shape[0]
    return pl.pallas_call(
        _pc_argmax_kernel,
        grid=(n // _BS,),
        in_specs=[
            pl.BlockSpec((_BS, _CELL_DIM), lambda i: (i, 0)),
            pl.BlockSpec((_NUM_CELLS, _CELL_DIM), lambda i: (0, 0)),
        ],
        out_specs=pl.BlockSpec((_BS,), lambda i: (i,)),
        out_shape=jax.ShapeDtypeStruct((n,), jnp.int32),
    )(states, placeCells)
